# Initial kernel scaffold; baseline (speedup 1.0000x reference)
import jax
import jax.numpy as jnp
from jax.experimental import pallas as pl


def kernel(z, W):
    raise NotImplementedError("write your pallas kernel here")


# speedup vs baseline: 6.6028x; 6.6028x over previous
import jax
import jax.numpy as jnp
from jax.experimental import pallas as pl
from jax.experimental.pallas import tpu as pltpu

_N_E = 8192
_E_DIM = 32
_BETA = 0.25
_N_TOK = 4096
_T = 256
_G = _N_TOK // _T


def _vq_body(zf_ref, wt_ref, w_ref,
             oh_ref, idx_ref, zq_ref, loss_ref, perp_ref,
             counts_ref, loss_acc_ref):
    i = pl.program_id(0)
    zf = zf_ref[...]
    wt = wt_ref[...]
    w = w_ref[...]

    zsq = jnp.sum(zf * zf, axis=1, keepdims=True)
    wsq = jnp.sum(w * w, axis=1)
    mm = jax.lax.dot_general(zf, wt, (((1,), (0,)), ((), ())),
                             preferred_element_type=jnp.float32)
    d = (zsq + wsq[None, :]) - 2.0 * mm

    m = jnp.min(d, axis=1, keepdims=True)
    iota = jax.lax.broadcasted_iota(jnp.int32, (_T, _N_E), 1)
    idx = jnp.min(jnp.where(d == m, iota, _N_E), axis=1, keepdims=True)

    oh = (iota == idx).astype(jnp.float32)
    oh_ref[...] = oh
    idx_ref[...] = idx

    zq = jax.lax.dot_general(oh, w, (((1,), (0,)), ((), ())),
                             preferred_element_type=jnp.float32)
    zq_ref[...] = zf + (zq - zf)

    diff = zq - zf
    part_loss = jnp.sum(diff * diff)
    part_counts = jnp.sum(oh, axis=0, keepdims=True)

    @pl.when(i == 0)
    def _():
        counts_ref[...] = part_counts
        loss_acc_ref[0] = part_loss

    @pl.when(i > 0)
    def _():
        counts_ref[...] += part_counts
        loss_acc_ref[0] += part_loss

    @pl.when(i == _G - 1)
    def _():
        mean = loss_acc_ref[0] / (_N_TOK * _E_DIM)
        loss_ref[...] = (mean + _BETA * mean).reshape(1, 1)
        e_mean = counts_ref[...] * (1.0 / _N_TOK)
        ent = jnp.sum(e_mean * jnp.log(e_mean + 1e-10))
        perp_ref[...] = jnp.exp(-ent).reshape(1, 1)


def kernel(z, W):
    zt = jnp.transpose(z, (0, 2, 3, 1))
    zf = zt.reshape(-1, _E_DIM)
    wt = W.T

    oh, idx, zq, loss, perp = pl.pallas_call(
        _vq_body,
        grid=(_G,),
        in_specs=[
            pl.BlockSpec((_T, _E_DIM), lambda i: (i, 0)),
            pl.BlockSpec((_E_DIM, _N_E), lambda i: (0, 0)),
            pl.BlockSpec((_N_E, _E_DIM), lambda i: (0, 0)),
        ],
        out_specs=[
            pl.BlockSpec((_T, _N_E), lambda i: (i, 0)),
            pl.BlockSpec((_T, 1), lambda i: (i, 0)),
            pl.BlockSpec((_T, _E_DIM), lambda i: (i, 0)),
            pl.BlockSpec((1, 1), lambda i: (0, 0)),
            pl.BlockSpec((1, 1), lambda i: (0, 0)),
        ],
        out_shape=[
            jax.ShapeDtypeStruct((_N_TOK, _N_E), jnp.float32),
            jax.ShapeDtypeStruct((_N_TOK, 1), jnp.int32),
            jax.ShapeDtypeStruct((_N_TOK, _E_DIM), jnp.float32),
            jax.ShapeDtypeStruct((1, 1), jnp.float32),
            jax.ShapeDtypeStruct((1, 1), jnp.float32),
        ],
        scratch_shapes=[
            pltpu.VMEM((1, _N_E), jnp.float32),
            pltpu.SMEM((1,), jnp.float32),
        ],
        compiler_params=pltpu.CompilerParams(
            dimension_semantics=("arbitrary",),
        ),
    )(zf, wt, W)

    z_q = jnp.transpose(zq.reshape(zt.shape), (0, 3, 1, 2))
    return (loss.reshape(()), z_q, perp.reshape(()), oh, idx)


# speedup vs baseline: 6.8748x; 1.0412x over previous
import jax
import jax.numpy as jnp
from jax.experimental import pallas as pl
from jax.experimental.pallas import tpu as pltpu

_N_E = 8192
_E_DIM = 32
_BETA = 0.25
_N_TOK = 4096
_T = 256
_G = _N_TOK // _T


def _vq_body(zf_ref, wt_ref, w_ref, iota_ref,
             oh_ref, idx_ref, zq_ref, loss_ref, perp_ref,
             counts_ref, loss_acc_ref):
    i = pl.program_id(0)
    zf = zf_ref[...]
    wt = wt_ref[...]
    w = w_ref[...]

    zsq = jnp.sum(zf * zf, axis=1, keepdims=True)
    wsq = jnp.sum(w * w, axis=1)
    mm2 = jax.lax.dot_general(zf + zf, wt, (((1,), (0,)), ((), ())),
                              preferred_element_type=jnp.float32)
    d = (zsq + wsq[None, :]) - mm2

    m = jnp.min(d, axis=1, keepdims=True)
    iota_f = iota_ref[...]
    idxf = jnp.min(jnp.where(d == m, iota_f, jnp.float32(_N_E)),
                   axis=1, keepdims=True)

    oh = (iota_f == idxf).astype(jnp.float32)
    oh_ref[...] = oh
    idx_ref[...] = idxf.astype(jnp.int32)

    zq = jax.lax.dot_general(oh, w, (((1,), (0,)), ((), ())),
                             preferred_element_type=jnp.float32)
    zq_ref[...] = zf + (zq - zf)

    diff = zq - zf
    part_loss = jnp.sum(diff * diff)
    part_counts = jnp.sum(oh, axis=0, keepdims=True)

    @pl.when(i == 0)
    def _():
        counts_ref[...] = part_counts
        loss_acc_ref[0] = part_loss

    @pl.when(i > 0)
    def _():
        counts_ref[...] += part_counts
        loss_acc_ref[0] += part_loss

    @pl.when(i == _G - 1)
    def _():
        mean = loss_acc_ref[0] / (_N_TOK * _E_DIM)
        loss_ref[...] = (mean + _BETA * mean).reshape(1, 1)
        e_mean = counts_ref[...] * (1.0 / _N_TOK)
        ent = jnp.sum(e_mean * jnp.log(e_mean + 1e-10))
        perp_ref[...] = jnp.exp(-ent).reshape(1, 1)


def kernel(z, W):
    zt = jnp.transpose(z, (0, 2, 3, 1))
    zf = zt.reshape(-1, _E_DIM)
    wt = W.T

    oh, idx, zq, loss, perp = pl.pallas_call(
        _vq_body,
        grid=(_G,),
        in_specs=[
            pl.BlockSpec((_T, _E_DIM), lambda i: (i, 0)),
            pl.BlockSpec((_E_DIM, _N_E), lambda i: (0, 0)),
            pl.BlockSpec((_N_E, _E_DIM), lambda i: (0, 0)),
            pl.BlockSpec((1, _N_E), lambda i: (0, 0)),
        ],
        out_specs=[
            pl.BlockSpec((_T, _N_E), lambda i: (i, 0)),
            pl.BlockSpec((_T, 1), lambda i: (i, 0)),
            pl.BlockSpec((_T, _E_DIM), lambda i: (i, 0)),
            pl.BlockSpec((1, 1), lambda i: (0, 0)),
            pl.BlockSpec((1, 1), lambda i: (0, 0)),
        ],
        out_shape=[
            jax.ShapeDtypeStruct((_N_TOK, _N_E), jnp.float32),
            jax.ShapeDtypeStruct((_N_TOK, 1), jnp.int32),
            jax.ShapeDtypeStruct((_N_TOK, _E_DIM), jnp.float32),
            jax.ShapeDtypeStruct((1, 1), jnp.float32),
            jax.ShapeDtypeStruct((1, 1), jnp.float32),
        ],
        scratch_shapes=[
            pltpu.VMEM((1, _N_E), jnp.float32),
            pltpu.SMEM((1,), jnp.float32),
        ],
        compiler_params=pltpu.CompilerParams(
            dimension_semantics=("arbitrary",),
        ),
    )(zf, wt, W, jnp.arange(_N_E, dtype=jnp.float32)[None, :])

    z_q = jnp.transpose(zq.reshape(zt.shape), (0, 3, 1, 2))
    return (loss.reshape(()), z_q, perp.reshape(()), oh, idx)
